# Initial kernel scaffold; baseline (speedup 1.0000x reference)
import functools

import jax
import jax.numpy as jnp
from jax import lax
from jax.experimental import pallas as pl
from jax.experimental.pallas import tpu as pltpu
from jax.experimental.pallas import tpu_sc as plsc

B, C, H, W = 8, 3, 512, 512
F = 20000
NC, NS, L = 2, 16, 16
NW = NC * NS
P = B * H * W
PPW = P // NW
ACC = F * C
CH = 2048
NCHUNK = PPW // CH
GROUPS = CH // L


def _sc_scatter_partials(pix_flat, img_flat):
    mesh = plsc.VectorSubcoreMesh(core_axis_name="c", subcore_axis_name="s")

    @functools.partial(
        pl.kernel,
        out_type=jax.ShapeDtypeStruct((NW, ACC), jnp.float32),
        mesh=mesh,
        scratch_types=[
            pltpu.VMEM((ACC,), jnp.float32),
            pltpu.VMEM((CH,), jnp.int32),
            pltpu.VMEM((CH,), jnp.float32),
            pltpu.VMEM((CH,), jnp.float32),
            pltpu.VMEM((CH,), jnp.float32),
        ],
    )
    def sc_kernel(pix_hbm, img_hbm, out_hbm, acc, idxb, vr, vg, vb):
        cid = lax.axis_index("c")
        sid = lax.axis_index("s")
        wid = sid * NC + cid
        b = wid // (NW // B)
        q = wid % (NW // B)
        shift = b * (3 * F)

        def zero_body(j, carry):
            acc[pl.ds(j * L, L)] = jnp.zeros((L,), jnp.float32)
            return carry

        lax.fori_loop(0, ACC // L, zero_body, 0)

        vbufs = (vr, vg, vb)
        for k in range(NCHUNK):
            pix_off = wid * PPW + k * CH
            pltpu.sync_copy(pix_hbm.at[pl.ds(pix_off, CH)], idxb)
            for c in range(3):
                img_off = (b * 3 + c) * (H * W) + q * PPW + k * CH
                pltpu.sync_copy(img_hbm.at[pl.ds(img_off, CH)], vbufs[c])

            def group_body(g, carry):
                ids = idxb[pl.ds(g * L, L)]
                i0 = ids * 3 - shift
                plsc.addupdate_scatter(acc, [i0], vr[pl.ds(g * L, L)])
                plsc.addupdate_scatter(acc, [i0 + 1], vg[pl.ds(g * L, L)])
                plsc.addupdate_scatter(acc, [i0 + 2], vb[pl.ds(g * L, L)])
                return carry

            lax.fori_loop(0, GROUPS, group_body, 0)

        pltpu.sync_copy(acc, out_hbm.at[wid])

    return sc_kernel(pix_flat, img_flat)


def _tc_reduce(partials):
    def body(x_ref, o_ref):
        o_ref[...] = jnp.sum(x_ref[...], axis=0)

    return pl.pallas_call(
        body,
        out_shape=jax.ShapeDtypeStruct((ACC,), jnp.float32),
    )(partials)


def kernel(images, vertices, faces, pix_to_face):
    del vertices, faces
    pix_flat = pix_to_face.reshape(-1)
    img_flat = images.reshape(-1)
    partials = _sc_scatter_partials(pix_flat, img_flat)
    colors = _tc_reduce(partials)
    return colors.reshape(F, C)


# speedup vs baseline: 24.5595x; 24.5595x over previous
import functools

import jax
import jax.numpy as jnp
from jax import lax
from jax.experimental import pallas as pl
from jax.experimental.pallas import tpu as pltpu
from jax.experimental.pallas import tpu_sc as plsc

B, C, H, W = 8, 3, 512, 512
F = 20000
NC, NS, L = 2, 16, 16
NW = NC * NS
P = B * H * W
PPW = P // NW
ACC = F * C
CH = 2048
NCHUNK = PPW // CH
GROUPS = CH // L


def _sc_scatter_partials(pix_flat, img_flat):
    mesh = plsc.VectorSubcoreMesh(core_axis_name="c", subcore_axis_name="s")

    @functools.partial(
        pl.kernel,
        out_type=jax.ShapeDtypeStruct((NW, ACC), jnp.float32),
        mesh=mesh,
        compiler_params=pltpu.CompilerParams(needs_layout_passes=False),
        scratch_types=[
            pltpu.VMEM((ACC,), jnp.float32),
            pltpu.VMEM((CH,), jnp.int32),
            pltpu.VMEM((CH,), jnp.float32),
            pltpu.VMEM((CH,), jnp.float32),
            pltpu.VMEM((CH,), jnp.float32),
        ],
    )
    def sc_kernel(pix_hbm, img_hbm, out_hbm, acc, idxb, vr, vg, vb):
        cid = lax.axis_index("c")
        sid = lax.axis_index("s")
        wid = sid * NC + cid
        b = wid // (NW // B)
        q = wid % (NW // B)
        shift = b * (3 * F)

        def zero_body(j, carry):
            acc[pl.ds(j * L, L)] = jnp.zeros((L,), jnp.float32)
            return carry

        lax.fori_loop(0, ACC // L, zero_body, 0)

        vbufs = (vr, vg, vb)
        for k in range(NCHUNK):
            pix_off = wid * PPW + k * CH
            pltpu.sync_copy(pix_hbm.at[pl.ds(pix_off, CH)], idxb)
            for c in range(3):
                img_off = (b * 3 + c) * (H * W) + q * PPW + k * CH
                pltpu.sync_copy(img_hbm.at[pl.ds(img_off, CH)], vbufs[c])

            def group_body(g, carry):
                ids = idxb[pl.ds(g * L, L)]
                i0 = ids * 3 - shift
                plsc.addupdate_scatter(acc, [i0], vr[pl.ds(g * L, L)])
                plsc.addupdate_scatter(acc, [i0 + 1], vg[pl.ds(g * L, L)])
                plsc.addupdate_scatter(acc, [i0 + 2], vb[pl.ds(g * L, L)])
                return carry

            lax.fori_loop(0, GROUPS, group_body, 0)

        pltpu.sync_copy(acc, out_hbm.at[wid])

    return sc_kernel(pix_flat, img_flat)


def _tc_reduce(partials):
    def body(x_ref, o_ref):
        o_ref[...] = jnp.sum(x_ref[...], axis=0)

    return pl.pallas_call(
        body,
        out_shape=jax.ShapeDtypeStruct((ACC,), jnp.float32),
    )(partials)


def kernel(images, vertices, faces, pix_to_face):
    del vertices, faces
    pix_flat = pix_to_face.reshape(-1)
    img_flat = images.reshape(-1)
    partials = _sc_scatter_partials(pix_flat, img_flat)
    colors = _tc_reduce(partials)
    return colors.reshape(F, C)


# speedup vs baseline: 37.3313x; 1.5200x over previous
import functools

import jax
import jax.numpy as jnp
from jax import lax
from jax.experimental import pallas as pl
from jax.experimental.pallas import tpu as pltpu
from jax.experimental.pallas import tpu_sc as plsc

B, C, H, W = 8, 3, 512, 512
F = 20000
NC, NS, L = 2, 16, 16
NW = NC * NS
ACC = F * C
RPW = H * B // NW
RCH = 8
NCHUNK = RPW // RCH
GPR = W // L


def _sc_scatter_partials(pix_to_face, images):
    mesh = plsc.VectorSubcoreMesh(core_axis_name="c", subcore_axis_name="s")

    @functools.partial(
        pl.kernel,
        out_type=jax.ShapeDtypeStruct((NW, ACC), jnp.float32),
        mesh=mesh,
        compiler_params=pltpu.CompilerParams(needs_layout_passes=False),
        scratch_types=[
            pltpu.VMEM((ACC,), jnp.float32),
            pltpu.VMEM((RCH, W), jnp.int32),
            pltpu.VMEM((RCH, W), jnp.float32),
            pltpu.VMEM((RCH, W), jnp.float32),
            pltpu.VMEM((RCH, W), jnp.float32),
        ],
    )
    def sc_kernel(pix_hbm, img_hbm, out_hbm, acc, idxb, vr, vg, vb):
        cid = lax.axis_index("c")
        sid = lax.axis_index("s")
        wid = sid * NC + cid
        b = wid // (NW // B)
        q = wid % (NW // B)
        shift = b * (3 * F)
        row0 = q * RPW

        def zero_body(j, carry):
            base = j * (10 * L)
            for u in range(10):
                acc[pl.ds(base + u * L, L)] = jnp.zeros((L,), jnp.float32)
            return carry

        lax.fori_loop(0, ACC // (10 * L), zero_body, 0)

        vbufs = (vr, vg, vb)

        def chunk_body(k, carry):
            r0 = row0 + k * RCH
            pltpu.sync_copy(pix_hbm.at[b, pl.ds(r0, RCH)], idxb)
            for c in range(3):
                pltpu.sync_copy(img_hbm.at[b, c, pl.ds(r0, RCH)], vbufs[c])

            def row_body(r, carry2):
                for u in range(GPR):
                    sl = pl.ds(u * L, L)
                    i0 = idxb[r, sl] * 3 - shift
                    plsc.addupdate_scatter(acc, [i0], vr[r, sl])
                    plsc.addupdate_scatter(acc, [i0 + 1], vg[r, sl])
                    plsc.addupdate_scatter(acc, [i0 + 2], vb[r, sl])
                return carry2

            lax.fori_loop(0, RCH, row_body, 0)
            return carry

        lax.fori_loop(0, NCHUNK, chunk_body, 0)

        pltpu.sync_copy(acc, out_hbm.at[wid])

    return sc_kernel(pix_to_face, images)


def _tc_reduce(partials):
    def body(x_ref, o_ref):
        o_ref[...] = jnp.sum(x_ref[...], axis=0)

    return pl.pallas_call(
        body,
        out_shape=jax.ShapeDtypeStruct((ACC,), jnp.float32),
    )(partials)


def kernel(images, vertices, faces, pix_to_face):
    del vertices, faces
    partials = _sc_scatter_partials(pix_to_face, images)
    colors = _tc_reduce(partials)
    return colors.reshape(F, C)


# speedup vs baseline: 49.9816x; 1.3389x over previous
import functools

import jax
import jax.numpy as jnp
from jax import lax
from jax.experimental import pallas as pl
from jax.experimental.pallas import tpu as pltpu
from jax.experimental.pallas import tpu_sc as plsc

B, C, H, W = 8, 3, 512, 512
F = 20000
NC, NS, L = 2, 16, 16
NW = NC * NS
ACC = F * C
RPW = H * B // NW
RCH = 8
NCHUNK = RPW // RCH
GPR = W // L


def _sc_scatter_partials(pix_to_face, images):
    mesh = plsc.VectorSubcoreMesh(core_axis_name="c", subcore_axis_name="s")

    @functools.partial(
        pl.kernel,
        out_type=jax.ShapeDtypeStruct((NW, ACC), jnp.float32),
        mesh=mesh,
        compiler_params=pltpu.CompilerParams(needs_layout_passes=False),
        scratch_types=[
            pltpu.VMEM((ACC,), jnp.float32),
            pltpu.VMEM((RCH, W), jnp.int32),
            pltpu.VMEM((RCH, W), jnp.float32),
            pltpu.VMEM((RCH, W), jnp.float32),
            pltpu.VMEM((RCH, W), jnp.float32),
        ],
    )
    def sc_kernel(pix_hbm, img_hbm, out_hbm, acc, idxb, vr, vg, vb):
        cid = lax.axis_index("c")
        sid = lax.axis_index("s")
        wid = sid * NC + cid
        b = wid // (NW // B)
        q = wid % (NW // B)
        shift = b * (3 * F)
        row0 = q * RPW

        @plsc.parallel_loop(0, ACC // L, unroll=8)
        def _zero(j):
            acc[pl.ds(j * L, L)] = jnp.zeros((L,), jnp.float32)

        vbufs = (vr, vg, vb)

        def chunk_body(k, carry):
            r0 = row0 + k * RCH
            pltpu.sync_copy(pix_hbm.at[b, pl.ds(r0, RCH)], idxb)
            for c in range(3):
                pltpu.sync_copy(img_hbm.at[b, c, pl.ds(r0, RCH)], vbufs[c])

            @plsc.parallel_loop(0, RCH * GPR, unroll=8)
            def _group(g):
                row = g >> 5
                sl = pl.ds((g & (GPR - 1)) * L, L)
                i0 = idxb[row, sl] * 3 - shift
                plsc.addupdate_scatter(acc, [i0], vr[row, sl])
                plsc.addupdate_scatter(acc, [i0 + 1], vg[row, sl])
                plsc.addupdate_scatter(acc, [i0 + 2], vb[row, sl])

            return carry

        lax.fori_loop(0, NCHUNK, chunk_body, 0)

        pltpu.sync_copy(acc, out_hbm.at[wid])

    return sc_kernel(pix_to_face, images)


def _tc_reduce(partials):
    def body(x_ref, o_ref):
        o_ref[...] = jnp.sum(x_ref[...], axis=0)

    return pl.pallas_call(
        body,
        out_shape=jax.ShapeDtypeStruct((ACC,), jnp.float32),
    )(partials)


def kernel(images, vertices, faces, pix_to_face):
    del vertices, faces
    partials = _sc_scatter_partials(pix_to_face, images)
    colors = _tc_reduce(partials)
    return colors.reshape(F, C)


# speedup vs baseline: 76.7638x; 1.5358x over previous
import functools

import jax
import jax.numpy as jnp
from jax import lax
from jax.experimental import pallas as pl
from jax.experimental.pallas import tpu as pltpu
from jax.experimental.pallas import tpu_sc as plsc

B, C, H, W = 8, 3, 512, 512
F = 20000
NC, NS, L = 2, 16, 16
NW = NC * NS
ACC = F * C
RPW = H * B // NW
RCH = 8
NCHUNK = RPW // RCH
GPR = W // L


def _sc_scatter_partials(pix_to_face, images):
    mesh = plsc.VectorSubcoreMesh(core_axis_name="c", subcore_axis_name="s")

    @functools.partial(
        pl.kernel,
        out_type=jax.ShapeDtypeStruct((NW, ACC), jnp.float32),
        mesh=mesh,
        compiler_params=pltpu.CompilerParams(needs_layout_passes=False),
        scratch_types=[
            pltpu.VMEM((ACC,), jnp.float32),
            pltpu.VMEM((RCH, W), jnp.int32),
            pltpu.VMEM((RCH, W), jnp.float32),
            pltpu.VMEM((RCH, W), jnp.float32),
            pltpu.VMEM((RCH, W), jnp.float32),
            pltpu.VMEM((RCH, W), jnp.int32),
            pltpu.VMEM((RCH, W), jnp.float32),
            pltpu.VMEM((RCH, W), jnp.float32),
            pltpu.VMEM((RCH, W), jnp.float32),
            pltpu.SemaphoreType.DMA,
            pltpu.SemaphoreType.DMA,
        ],
    )
    def sc_kernel(pix_hbm, img_hbm, out_hbm, acc,
                  idxb0, vr0, vg0, vb0, idxb1, vr1, vg1, vb1, sem0, sem1):
        cid = lax.axis_index("c")
        sid = lax.axis_index("s")
        wid = sid * NC + cid
        b = wid // (NW // B)
        q = wid % (NW // B)
        shift = b * (3 * F)
        row0 = q * RPW

        sets = ((idxb0, vr0, vg0, vb0, sem0), (idxb1, vr1, vg1, vb1, sem1))

        def start(k, bufs):
            idxb, vr, vg, vb, sem = bufs
            r0 = row0 + k * RCH
            pltpu.async_copy(pix_hbm.at[b, pl.ds(r0, RCH)], idxb, sem)
            for c, vbuf in ((0, vr), (1, vg), (2, vb)):
                pltpu.async_copy(img_hbm.at[b, c, pl.ds(r0, RCH)], vbuf, sem)

        def drain(bufs):
            idxb, vr, vg, vb, sem = bufs
            pltpu.make_async_copy(pix_hbm.at[b, pl.ds(row0, RCH)], idxb, sem).wait()
            for c, vbuf in ((0, vr), (1, vg), (2, vb)):
                pltpu.make_async_copy(
                    img_hbm.at[b, c, pl.ds(row0, RCH)], vbuf, sem).wait()

        @plsc.parallel_loop(0, ACC // L, unroll=8)
        def _zero(j):
            acc[pl.ds(j * L, L)] = jnp.zeros((L,), jnp.float32)

        start(0, sets[0])
        for k in range(NCHUNK):
            idxb, vr, vg, vb, _ = bufs = sets[k % 2]
            if k + 1 < NCHUNK:
                start(k + 1, sets[(k + 1) % 2])
            drain(bufs)

            @plsc.parallel_loop(0, RCH * GPR, unroll=8)
            def _group(g):
                row = g >> 5
                sl = pl.ds((g & (GPR - 1)) * L, L)
                i0 = idxb[row, sl] * 3 - shift
                plsc.addupdate_scatter(acc, [i0], vr[row, sl])
                plsc.addupdate_scatter(acc, [i0 + 1], vg[row, sl])
                plsc.addupdate_scatter(acc, [i0 + 2], vb[row, sl])

        pltpu.sync_copy(acc, out_hbm.at[wid])

    return sc_kernel(pix_to_face, images)


def _tc_reduce(partials):
    def body(x_ref, o_ref):
        o_ref[...] = jnp.sum(x_ref[...], axis=0)

    return pl.pallas_call(
        body,
        out_shape=jax.ShapeDtypeStruct((ACC,), jnp.float32),
    )(partials)


def kernel(images, vertices, faces, pix_to_face):
    del vertices, faces
    partials = _sc_scatter_partials(pix_to_face, images)
    colors = _tc_reduce(partials)
    return colors.reshape(F, C)
